# baseline (speedup 1.0000x reference)
import functools

import jax
import jax.numpy as jnp
from jax import lax
from jax.experimental import pallas as pl
from jax.experimental.pallas import tpu as pltpu
from jax.experimental.pallas import tpu_sc as plsc

N_NODES = 100000
C_DIM = 128
KVOL = 27
E_PER = 20000
E = KVOL * E_PER
E_PAD = 540160
NB_TOT = E_PAD // 128
OM_PAD = 557056
EPW = OM_PAD // 16

N_CHUNK = 8
CH = 12544
ACC_ROWS = 12608
DUMMY_ROW = 12544
ZROWS = CH // 16
LAST_REAL = N_NODES - (N_CHUNK - 1) * CH
SENTINEL = 1 << 28

_mesh = lambda: plsc.VectorSubcoreMesh(core_axis_name="c", subcore_axis_name="s")
_sc_params = lambda: pltpu.CompilerParams(needs_layout_passes=False)


def _sc_gather(x, im):
    @functools.partial(
        pl.kernel,
        out_type=jax.ShapeDtypeStruct((E_PAD, C_DIM), jnp.float32),
        mesh=_mesh(),
        scratch_types=[
            pltpu.VMEM((2, 128), jnp.int32),
            pltpu.VMEM((2, 128, C_DIM), jnp.float32),
            pltpu.SemaphoreType.DMA((2,)),
            pltpu.SemaphoreType.DMA((2,)),
            pltpu.SemaphoreType.DMA((2,)),
        ],
        compiler_params=_sc_params(),
    )
    def k(x_hbm, im_hbm, g_hbm, idxs, rows, isem, gsem, wsem):
        w = lax.axis_index("s") * 2 + lax.axis_index("c")
        nb = jnp.where(w < NB_TOT - 32 * (NB_TOT // 32), NB_TOT // 32 + 1,
                       NB_TOT // 32)

        def boff(t):
            return (w + 32 * t) * 128

        def start_idx(t):
            s = lax.rem(t, 2)
            pltpu.async_copy(im_hbm.at[pl.ds(boff(t), 128)], idxs.at[s],
                             isem.at[s])

        def wait_idx(t):
            s = lax.rem(t, 2)
            pltpu.make_async_copy(im_hbm.at[pl.ds(boff(t), 128)], idxs.at[s],
                                  isem.at[s]).wait()

        def start_gather(t):
            s = lax.rem(t, 2)
            pltpu.async_copy(x_hbm.at[idxs.at[s]], rows.at[s], gsem.at[s])

        def wait_gather(t):
            s = lax.rem(t, 2)
            pltpu.make_async_copy(x_hbm.at[idxs.at[s]], rows.at[s],
                                  gsem.at[s]).wait()

        def start_wb(t):
            s = lax.rem(t, 2)
            pltpu.async_copy(rows.at[s], g_hbm.at[pl.ds(boff(t), 128)],
                             wsem.at[s])

        def wait_wb(t):
            s = lax.rem(t, 2)
            pltpu.make_async_copy(rows.at[s], g_hbm.at[pl.ds(boff(t), 128)],
                                  wsem.at[s]).wait()

        start_idx(0)

        def it(t, carry):
            @pl.when(t >= 2)
            def _():
                wait_wb(t - 2)

            @pl.when(t >= 1)
            def _():
                wait_gather(t - 1)

            @pl.when(t + 1 < nb)
            def _():
                start_idx(t + 1)

            @pl.when(t >= 1)
            def _():
                start_wb(t - 1)

            wait_idx(t)
            start_gather(t)
            return carry

        lax.fori_loop(0, nb, it, 0)
        wait_gather(nb - 1)
        start_wb(nb - 1)

        @pl.when(nb >= 2)
        def _():
            wait_wb(nb - 2)

        wait_wb(nb - 1)

    return k(x, im)


def _tc_matmul(g, wts):
    def mm(g_ref, w_ref, o_ref):
        o_ref[...] = jnp.dot(g_ref[...], w_ref[0], preferred_element_type=jnp.float32)

    blk = 2000
    return pl.pallas_call(
        mm,
        grid=(KVOL, E_PER // blk),
        in_specs=[
            pl.BlockSpec((blk, C_DIM), lambda k, e: (k * (E_PER // blk) + e, 0)),
            pl.BlockSpec((1, C_DIM, C_DIM), lambda k, e: (k, 0, 0)),
        ],
        out_specs=pl.BlockSpec((blk, C_DIM), lambda k, e: (k * (E_PER // blk) + e, 0)),
        out_shape=jax.ShapeDtypeStruct((E_PAD, C_DIM), jnp.float32),
    )(g, wts)


def _sc_scatter(contrib, om, zrows):
    @functools.partial(
        pl.kernel,
        out_type=jax.ShapeDtypeStruct((N_NODES, C_DIM), jnp.float32),
        mesh=_mesh(),
        scratch_types=[
            pltpu.VMEM((2, 1024), jnp.int32),
            pltpu.VMEM((512,), jnp.int32),
            pltpu.VMEM((512,), jnp.int32),
            pltpu.VMEM((3, 64), jnp.int32),
            pltpu.VMEM((3, 64), jnp.int32),
            pltpu.VMEM((3, 64, C_DIM), jnp.float32),
            pltpu.VMEM_SHARED((ACC_ROWS, C_DIM), jnp.float32),
            pltpu.SemaphoreType.DMA((2,)),
            pltpu.SemaphoreType.DMA((3,)),
            pltpu.SemaphoreType.DMA((3,)),
            pltpu.SemaphoreType.DMA,
        ],
        compiler_params=_sc_params(),
    )
    def k(ct_hbm, om_hbm, z_hbm, out_hbm, om2, idbuf, locbuf, idst, locst,
          rows, acc, osem, gsem, ssem, zsem):
        cid = lax.axis_index("c")
        sid = lax.axis_index("s")
        iota16 = lax.iota(jnp.int32, 16)

        wbase = sid * EPW
        zbase = sid * ZROWS

        def do_pass(p, carry):
            c = cid + 2 * p
            lo = c * CH

            pltpu.async_copy(z_hbm, acc.at[pl.ds(zbase, ZROWS)], zsem)
            pltpu.make_async_copy(z_hbm, acc.at[pl.ds(zbase, ZROWS)],
                                  zsem).wait()
            plsc.subcore_barrier()

            def stage(slot, q):
                base = 64 * lax.rem(q, 8)
                for j in range(4):
                    idst[slot, pl.ds(16 * j, 16)] = idbuf[pl.ds(base + 16 * j, 16)]
                    locst[slot, pl.ds(16 * j, 16)] = locbuf[pl.ds(base + 16 * j, 16)]

            def start_gather(slot):
                pltpu.async_copy(ct_hbm.at[idst.at[slot]], rows.at[slot],
                                 gsem.at[slot])

            def wait_gather(slot):
                pltpu.make_async_copy(ct_hbm.at[idst.at[slot]], rows.at[slot],
                                      gsem.at[slot]).wait()

            def start_scat(slot):
                pltpu.async_copy(rows.at[slot], acc.at[locst.at[slot]],
                                 ssem.at[slot], add=True)

            def wait_scat(slot):
                pltpu.make_async_copy(rows.at[slot], acc.at[locst.at[slot]],
                                      ssem.at[slot]).wait()

            def fire(q):
                s = lax.rem(q, 3)

                @pl.when(q >= 3)
                def _():
                    wait_scat(s)

                stage(s, q)
                start_gather(s)

                @pl.when(q >= 1)
                def _():
                    sp = lax.rem(q - 1, 3)
                    wait_gather(sp)
                    start_scat(sp)

            def scan_body(i, carry2, slot, off):
                cnt, q = carry2
                v = om2[slot, pl.ds(16 * i, 16)]
                m = (v >= lo) & (v < lo + CH)
                pc = plsc.all_reduce_population_count(m)[0]

                @pl.when(pc > 0)
                def _():
                    ids = (off + 16 * i) + iota16
                    mi = m.astype(jnp.int32)
                    pos = (cnt + plsc.cumsum(mi) - 1) & 511
                    plsc.store_scatter(idbuf, [pos], ids, mask=m)
                    plsc.store_scatter(locbuf, [pos], v - lo, mask=m)

                cnt = cnt + pc
                full = cnt - 64 * q >= 64

                @pl.when(full)
                def _():
                    fire(q)

                return cnt, jnp.where(full, q + 1, q)

            def start_om(b):
                s = lax.rem(b, 2)
                pltpu.async_copy(om_hbm.at[pl.ds(wbase + b * 1024, 1024)],
                                 om2.at[s], osem.at[s])

            def wait_om(b):
                s = lax.rem(b, 2)
                pltpu.make_async_copy(
                    om_hbm.at[pl.ds(wbase + b * 1024, 1024)], om2.at[s],
                    osem.at[s]).wait()

            start_om(0)

            def quad(i, carry2, slot, off):
                a = carry2
                for u in range(4):
                    a = scan_body(4 * i + u, a, slot, off)
                return a

            def blk(b, carry2):
                @pl.when(b + 1 < 34)
                def _():
                    start_om(b + 1)

                wait_om(b)
                s = lax.rem(b, 2)
                return lax.fori_loop(
                    0, 16,
                    lambda i, a: quad(i, a, s, wbase + b * 1024), carry2)

            cnt, q = lax.fori_loop(0, 34, blk, (jnp.int32(0), jnp.int32(0)))

            rem = cnt - 64 * q

            @pl.when(rem > 0)
            def _():
                for j in range(4):
                    ppos = (cnt + 16 * j + iota16) & 511
                    plsc.store_scatter(idbuf, [ppos], jnp.zeros((16,), jnp.int32))
                    plsc.store_scatter(locbuf, [ppos],
                                       jnp.full((16,), DUMMY_ROW, jnp.int32))
                fire(q)

            nq = jnp.where(rem > 0, q + 1, q)

            @pl.when(nq >= 1)
            def _():
                sl = lax.rem(nq - 1, 3)
                wait_gather(sl)
                start_scat(sl)

            for j in range(3):
                @pl.when(nq - 1 - j >= 0)
                def _(j=j):
                    wait_scat(lax.rem(nq - 1 - j, 3))

            plsc.subcore_barrier()

            is_last = c == N_CHUNK - 1

            @pl.when(sid < 15)
            def _():
                pltpu.sync_copy(acc.at[pl.ds(sid * ZROWS, ZROWS)],
                                out_hbm.at[pl.ds(lo + sid * ZROWS, ZROWS)])

            @pl.when((sid == 15) & jnp.logical_not(is_last))
            def _():
                pltpu.sync_copy(acc.at[pl.ds(15 * ZROWS, ZROWS)],
                                out_hbm.at[pl.ds(lo + 15 * ZROWS, ZROWS)])

            @pl.when((sid == 15) & is_last)
            def _():
                pltpu.sync_copy(
                    acc.at[pl.ds(15 * ZROWS, LAST_REAL - 15 * ZROWS)],
                    out_hbm.at[pl.ds(lo + 15 * ZROWS, LAST_REAL - 15 * ZROWS)])

            plsc.subcore_barrier()
            return carry

        lax.fori_loop(0, 4, do_pass, 0)

    return k(contrib, om, zrows)


def kernel(x, kernel, in_map, out_map):
    wts = kernel
    im = jnp.concatenate(
        [in_map.reshape(-1).astype(jnp.int32),
         jnp.zeros((E_PAD - E,), jnp.int32)])
    om = jnp.concatenate(
        [out_map.reshape(-1).astype(jnp.int32),
         jnp.full((OM_PAD - E,), SENTINEL, jnp.int32)])
    z = jnp.zeros((ZROWS, C_DIM), jnp.float32)
    gathered = _sc_gather(x, im)
    contrib = _tc_matmul(gathered, wts)
    return _sc_scatter(contrib, om, z)


# speedup vs baseline: 1.0749x; 1.0749x over previous
import functools

import jax
import jax.numpy as jnp
from jax import lax
from jax.experimental import pallas as pl
from jax.experimental.pallas import tpu as pltpu
from jax.experimental.pallas import tpu_sc as plsc

N_NODES = 100000
C_DIM = 128
KVOL = 27
E_PER = 20000
E = KVOL * E_PER
KV_A = 14
E_A = KV_A * E_PER
E_B = E - E_A
EA_PAD = 280064
EB_PAD = 260096
OM_PAD = 557056
EPW = OM_PAD // 16

N_CHUNK = 8
CH = 12544
ACC_ROWS = 12608
DUMMY_ROW = 12544
ZROWS = CH // 16
LAST_REAL = N_NODES - (N_CHUNK - 1) * CH
SENTINEL = 1 << 28

_mesh = lambda: plsc.VectorSubcoreMesh(core_axis_name="c", subcore_axis_name="s")
_sc_params = lambda: pltpu.CompilerParams(needs_layout_passes=False)


def _sc_gather(x, im, n_pad):
    nb_tot = n_pad // 128

    @functools.partial(
        pl.kernel,
        out_type=jax.ShapeDtypeStruct((n_pad, C_DIM), jnp.float32),
        mesh=_mesh(),
        scratch_types=[
            pltpu.VMEM((2, 128), jnp.int32),
            pltpu.VMEM((2, 128, C_DIM), jnp.float32),
            pltpu.SemaphoreType.DMA((2,)),
            pltpu.SemaphoreType.DMA((2,)),
            pltpu.SemaphoreType.DMA((2,)),
        ],
        compiler_params=_sc_params(),
    )
    def k(x_hbm, im_hbm, g_hbm, idxs, rows, isem, gsem, wsem):
        w = lax.axis_index("s") * 2 + lax.axis_index("c")
        nb = jnp.where(w < nb_tot - 32 * (nb_tot // 32), nb_tot // 32 + 1,
                       nb_tot // 32)

        def boff(t):
            return (w + 32 * t) * 128

        def start_idx(t):
            s = lax.rem(t, 2)
            pltpu.async_copy(im_hbm.at[pl.ds(boff(t), 128)], idxs.at[s],
                             isem.at[s])

        def wait_idx(t):
            s = lax.rem(t, 2)
            pltpu.make_async_copy(im_hbm.at[pl.ds(boff(t), 128)],
                                  idxs.at[s], isem.at[s]).wait()

        def start_gather(t):
            s = lax.rem(t, 2)
            pltpu.async_copy(x_hbm.at[idxs.at[s]], rows.at[s], gsem.at[s])

        def wait_gather(t):
            s = lax.rem(t, 2)
            pltpu.make_async_copy(x_hbm.at[idxs.at[s]], rows.at[s],
                                  gsem.at[s]).wait()

        def start_wb(t):
            s = lax.rem(t, 2)
            pltpu.async_copy(rows.at[s], g_hbm.at[pl.ds(boff(t), 128)],
                             wsem.at[s])

        def wait_wb(t):
            s = lax.rem(t, 2)
            pltpu.make_async_copy(rows.at[s], g_hbm.at[pl.ds(boff(t), 128)],
                                  wsem.at[s]).wait()

        start_idx(0)

        def it(t, carry):
            @pl.when(t >= 2)
            def _():
                wait_wb(t - 2)

            @pl.when(t >= 1)
            def _():
                wait_gather(t - 1)

            @pl.when(t + 1 < nb)
            def _():
                start_idx(t + 1)

            @pl.when(t >= 1)
            def _():
                start_wb(t - 1)

            wait_idx(t)
            start_gather(t)
            return carry

        lax.fori_loop(0, nb, it, 0)
        wait_gather(nb - 1)
        start_wb(nb - 1)
        wait_wb(nb - 2)
        wait_wb(nb - 1)

    return k(x, im)


_BLK = 2000
_EB = E_PER // _BLK


def _tc_matmul_a(g, wts):
    def mm(g_ref, w_ref, o_ref):
        o_ref[...] = jnp.dot(g_ref[...], w_ref[0], preferred_element_type=jnp.float32)

    return pl.pallas_call(
        mm,
        grid=(KV_A, _EB),
        in_specs=[
            pl.BlockSpec((_BLK, C_DIM), lambda k, e: (k * _EB + e, 0)),
            pl.BlockSpec((1, C_DIM, C_DIM), lambda k, e: (k, 0, 0)),
        ],
        out_specs=pl.BlockSpec((_BLK, C_DIM), lambda k, e: (k * _EB + e, 0)),
        out_shape=jax.ShapeDtypeStruct((E, C_DIM), jnp.float32),
    )(g, wts)


def _tc_matmul_b(g, wts, acc):
    def mm(g_ref, w_ref, a_ref, o_ref):
        o_ref[...] = jnp.dot(g_ref[...], w_ref[0], preferred_element_type=jnp.float32)

    base = E_A // _BLK
    return pl.pallas_call(
        mm,
        grid=(KVOL - KV_A, _EB),
        in_specs=[
            pl.BlockSpec((_BLK, C_DIM), lambda k, e: (k * _EB + e, 0)),
            pl.BlockSpec((1, C_DIM, C_DIM), lambda k, e: (k + KV_A, 0, 0)),
            pl.BlockSpec(memory_space=pl.ANY),
        ],
        out_specs=pl.BlockSpec((_BLK, C_DIM), lambda k, e: (base + k * _EB + e, 0)),
        out_shape=jax.ShapeDtypeStruct((E, C_DIM), jnp.float32),
        input_output_aliases={2: 0},
    )(g, wts, acc)


def _sc_scatter(contrib, om, zrows):
    @functools.partial(
        pl.kernel,
        out_type=jax.ShapeDtypeStruct((N_NODES, C_DIM), jnp.float32),
        mesh=_mesh(),
        scratch_types=[
            pltpu.VMEM((2, 1024), jnp.int32),
            pltpu.VMEM((512,), jnp.int32),
            pltpu.VMEM((512,), jnp.int32),
            pltpu.VMEM((3, 64), jnp.int32),
            pltpu.VMEM((3, 64), jnp.int32),
            pltpu.VMEM((3, 64, C_DIM), jnp.float32),
            pltpu.VMEM_SHARED((ACC_ROWS, C_DIM), jnp.float32),
            pltpu.SemaphoreType.DMA((2,)),
            pltpu.SemaphoreType.DMA((3,)),
            pltpu.SemaphoreType.DMA((3,)),
            pltpu.SemaphoreType.DMA,
        ],
        compiler_params=_sc_params(),
    )
    def k(ct_hbm, om_hbm, z_hbm, out_hbm, om2, idbuf, locbuf, idst, locst,
          rows, acc, osem, gsem, ssem, zsem):
        cid = lax.axis_index("c")
        sid = lax.axis_index("s")
        iota16 = lax.iota(jnp.int32, 16)

        wbase = sid * EPW
        zbase = sid * ZROWS

        def do_pass(p, carry):
            c = cid + 2 * p
            lo = c * CH

            pltpu.async_copy(z_hbm, acc.at[pl.ds(zbase, ZROWS)], zsem)
            pltpu.make_async_copy(z_hbm, acc.at[pl.ds(zbase, ZROWS)],
                                  zsem).wait()
            plsc.subcore_barrier()

            def stage(slot, q):
                base = 64 * lax.rem(q, 8)
                for j in range(4):
                    idst[slot, pl.ds(16 * j, 16)] = idbuf[pl.ds(base + 16 * j, 16)]
                    locst[slot, pl.ds(16 * j, 16)] = locbuf[pl.ds(base + 16 * j, 16)]

            def start_gather(slot):
                pltpu.async_copy(ct_hbm.at[idst.at[slot]], rows.at[slot],
                                 gsem.at[slot])

            def wait_gather(slot):
                pltpu.make_async_copy(ct_hbm.at[idst.at[slot]], rows.at[slot],
                                      gsem.at[slot]).wait()

            def start_scat(slot):
                pltpu.async_copy(rows.at[slot], acc.at[locst.at[slot]],
                                 ssem.at[slot], add=True)

            def wait_scat(slot):
                pltpu.make_async_copy(rows.at[slot], acc.at[locst.at[slot]],
                                      ssem.at[slot]).wait()

            def fire(q):
                s = lax.rem(q, 3)

                @pl.when(q >= 3)
                def _():
                    wait_scat(s)

                stage(s, q)
                start_gather(s)

                @pl.when(q >= 1)
                def _():
                    sp = lax.rem(q - 1, 3)
                    wait_gather(sp)
                    start_scat(sp)

            def scan_body(i, carry2, slot, off):
                cnt, q = carry2
                v = om2[slot, pl.ds(16 * i, 16)]
                m = (v >= lo) & (v < lo + CH)
                pc = plsc.all_reduce_population_count(m)[0]

                @pl.when(pc > 0)
                def _():
                    ids = (off + 16 * i) + iota16
                    mi = m.astype(jnp.int32)
                    pos = (cnt + plsc.cumsum(mi) - 1) & 511
                    plsc.store_scatter(idbuf, [pos], ids, mask=m)
                    plsc.store_scatter(locbuf, [pos], v - lo, mask=m)

                cnt = cnt + pc
                full = cnt - 64 * q >= 64

                @pl.when(full)
                def _():
                    fire(q)

                return cnt, jnp.where(full, q + 1, q)

            def start_om(b):
                s = lax.rem(b, 2)
                pltpu.async_copy(om_hbm.at[pl.ds(wbase + b * 1024, 1024)],
                                 om2.at[s], osem.at[s])

            def wait_om(b):
                s = lax.rem(b, 2)
                pltpu.make_async_copy(
                    om_hbm.at[pl.ds(wbase + b * 1024, 1024)], om2.at[s],
                    osem.at[s]).wait()

            start_om(0)

            def blk(b, carry2):
                @pl.when(b + 1 < 34)
                def _():
                    start_om(b + 1)

                wait_om(b)
                s = lax.rem(b, 2)
                return lax.fori_loop(
                    0, 64,
                    lambda i, a: scan_body(i, a, s, wbase + b * 1024), carry2)

            cnt, q = lax.fori_loop(0, 34, blk, (jnp.int32(0), jnp.int32(0)))

            rem = cnt - 64 * q

            @pl.when(rem > 0)
            def _():
                for j in range(4):
                    ppos = (cnt + 16 * j + iota16) & 511
                    plsc.store_scatter(idbuf, [ppos], jnp.zeros((16,), jnp.int32))
                    plsc.store_scatter(locbuf, [ppos],
                                       jnp.full((16,), DUMMY_ROW, jnp.int32))
                fire(q)

            nq = jnp.where(rem > 0, q + 1, q)

            @pl.when(nq >= 1)
            def _():
                sl = lax.rem(nq - 1, 3)
                wait_gather(sl)
                start_scat(sl)

            for j in range(3):
                @pl.when(nq - 1 - j >= 0)
                def _(j=j):
                    wait_scat(lax.rem(nq - 1 - j, 3))

            plsc.subcore_barrier()

            is_last = c == N_CHUNK - 1

            @pl.when(sid < 15)
            def _():
                pltpu.sync_copy(acc.at[pl.ds(sid * ZROWS, ZROWS)],
                                out_hbm.at[pl.ds(lo + sid * ZROWS, ZROWS)])

            @pl.when((sid == 15) & jnp.logical_not(is_last))
            def _():
                pltpu.sync_copy(acc.at[pl.ds(15 * ZROWS, ZROWS)],
                                out_hbm.at[pl.ds(lo + 15 * ZROWS, ZROWS)])

            @pl.when((sid == 15) & is_last)
            def _():
                pltpu.sync_copy(
                    acc.at[pl.ds(15 * ZROWS, LAST_REAL - 15 * ZROWS)],
                    out_hbm.at[pl.ds(lo + 15 * ZROWS, LAST_REAL - 15 * ZROWS)])

            plsc.subcore_barrier()
            return carry

        lax.fori_loop(0, 4, do_pass, 0)

    return k(contrib, om, zrows)


def kernel(x, kernel, in_map, out_map):
    wts = kernel
    im = in_map.reshape(-1).astype(jnp.int32)
    im_a = jnp.concatenate([im[:E_A], jnp.zeros((EA_PAD - E_A,), jnp.int32)])
    im_b = jnp.concatenate([im[E_A:], jnp.zeros((EB_PAD - E_B,), jnp.int32)])
    om = jnp.concatenate(
        [out_map.reshape(-1).astype(jnp.int32),
         jnp.full((OM_PAD - E,), SENTINEL, jnp.int32)])
    z = jnp.zeros((ZROWS, C_DIM), jnp.float32)
    g_a = _sc_gather(x, im_a, EA_PAD)
    c_a = _tc_matmul_a(g_a, wts)
    g_b = _sc_gather(x, im_b, EB_PAD)
    contrib = _tc_matmul_b(g_b, wts, c_a)
    return _sc_scatter(contrib, om, z)
